# Initial kernel scaffold; baseline (speedup 1.0000x reference)
import jax
import jax.numpy as jnp
from jax.experimental import pallas as pl


def kernel(x, edge_index, W1_l, b1_l, W1_r, W2_l, b2_l, W2_r):
    raise NotImplementedError("write your pallas kernel here")


# speedup vs baseline: 6.8040x; 6.8040x over previous
import functools

import jax
import jax.numpy as jnp
from jax import lax
from jax.experimental import pallas as pl
from jax.experimental.pallas import tpu as pltpu
from jax.experimental.pallas import tpu_sc as plsc

_N = 10000
_NPAD = 10240
_D = 128
_E = 320000
_NC = 2
_NS = 16
_NW = _NC * _NS
_CH = 128
_EW = 10240
_EPAD = _NW * _EW
_NCHUNK = _EW // _CH
_G = 8
_NG = _NCHUNK // _G
_RPT = _NPAD // _NS
_DEGW = 16
_DAUG = _D + _DEGW


def _zero_rows(rows, width):
  zv = jnp.zeros((16,), jnp.float32)
  def zrow(i, carry):
    for k in range(width // 16):
      rows[i, pl.ds(k * 16, 16)] = zv
    return carry
  lax.fori_loop(0, _CH, zrow, 0)


def _make_segsum(compute_deg):
  mesh = plsc.VectorSubcoreMesh(core_axis_name="c", subcore_axis_name="s")
  out_type = jax.ShapeDtypeStruct((_NC, _NPAD, _D), jnp.float32)
  scratch = (
      pltpu.VMEM_SHARED((_NPAD, _D), jnp.float32),
      pltpu.VMEM((_G, _CH), jnp.int32),
      pltpu.VMEM((_G, _CH), jnp.int32),
      pltpu.VMEM((_CH, _D), jnp.float32),
      pltpu.SemaphoreType.DMA,
  )

  def body(feat, srcr, dstr, s_out, acc, srcv, dstv, rows, sem):
    c = lax.axis_index("c")
    s = lax.axis_index("s")
    w = c * _NS + s
    r0 = s * _RPT
    nblk = _RPT // _CH

    _zero_rows(rows, _D)
    for t in range(nblk):
      pltpu.sync_copy(rows, acc.at[pl.ds(r0 + t * _CH, _CH)])
    if compute_deg:
      ov = jnp.ones((16,), jnp.float32)
      def onerow(i, carry):
        for k in range(_D // 16):
          rows[i, pl.ds(k * 16, 16)] = ov
        return carry
      lax.fori_loop(0, _CH, onerow, 0)
    plsc.subcore_barrier()

    def group(g, carry):
      if not compute_deg:
        pltpu.sync_copy(srcr.at[w, pl.ds(g * _G, _G)], srcv)
      pltpu.sync_copy(dstr.at[w, pl.ds(g * _G, _G)], dstv)
      for j in range(_G):
        if not compute_deg:
          pltpu.async_copy(feat.at[srcv.at[j]], rows, sem).wait()
        pltpu.sync_copy(rows, acc.at[dstv.at[j]], add=True)
      return carry

    lax.fori_loop(0, _NG, group, 0)
    plsc.subcore_barrier()
    for t in range(nblk):
      rb = r0 + t * _CH
      pltpu.sync_copy(acc.at[pl.ds(rb, _CH)], rows)
      pltpu.sync_copy(rows, s_out.at[c, pl.ds(rb, _CH)])

  return functools.partial(
      pl.kernel, mesh=mesh, out_type=out_type,
      scratch_types=scratch)(body)


_make_segsum = functools.lru_cache(maxsize=None)(_make_segsum)

_BN = 1024


def _dense_layer(S, deg, x, W_l, b_l, W_r, *, sigmoid):
  def body(s_ref, d_ref, x_ref, wl_ref, b_ref, wr_ref, o_ref):
    ssum = s_ref[0] + s_ref[1]
    agg = ssum / jnp.maximum(d_ref[0] + d_ref[1], 1.0)
    r = jnp.dot(agg, wl_ref[...], preferred_element_type=jnp.float32,
                precision=lax.Precision.HIGHEST)
    r += jnp.dot(x_ref[...], wr_ref[...], preferred_element_type=jnp.float32,
                 precision=lax.Precision.HIGHEST)
    r += b_ref[...]
    o_ref[...] = jax.nn.sigmoid(r) if sigmoid else jnp.maximum(r, 0.0)

  return pl.pallas_call(
      body,
      grid=(_NPAD // _BN,),
      in_specs=[
          pl.BlockSpec((2, _BN, _D), lambda i: (0, i, 0)),
          pl.BlockSpec((2, _BN, _D), lambda i: (0, i, 0)),
          pl.BlockSpec((_BN, _D), lambda i: (i, 0)),
          pl.BlockSpec((_D, _D), lambda i: (0, 0)),
          pl.BlockSpec((1, _D), lambda i: (0, 0)),
          pl.BlockSpec((_D, _D), lambda i: (0, 0)),
      ],
      out_specs=pl.BlockSpec((_BN, _D), lambda i: (i, 0)),
      out_shape=jax.ShapeDtypeStruct((_NPAD, _D), jnp.float32),
  )(S, deg, x, W_l, b_l.reshape(1, _D), W_r)


@jax.jit
def kernel(x, edge_index, W1_l, b1_l, W1_r, W2_l, b2_l, W2_r):
  src = edge_index[0]
  dst = edge_index[1]
  npad = _EPAD - _E
  pad_src = (jnp.arange(npad, dtype=jnp.int32) * 37) % _N
  pad_dst = _N + jnp.arange(npad, dtype=jnp.int32) % (_NPAD - _N)
  srcp = jnp.concatenate([src, pad_src]).reshape(_NW, _NCHUNK, _CH)
  dstp = jnp.concatenate([dst, pad_dst]).reshape(_NW, _NCHUNK, _CH)
  xpad = jnp.concatenate([x, jnp.zeros((_NPAD - _N, _D), x.dtype)])

  deg = _make_segsum(True)(xpad, srcp, dstp)
  S1 = _make_segsum(False)(xpad, srcp, dstp)
  h = _dense_layer(S1, deg, xpad, W1_l, b1_l, W1_r, sigmoid=False)
  S2 = _make_segsum(False)(h, srcp, dstp)
  out = _dense_layer(S2, deg, h, W2_l, b2_l, W2_r, sigmoid=True)
  return out[:_N]


# speedup vs baseline: 8.6753x; 1.2750x over previous
import functools

import jax
import jax.numpy as jnp
from jax import lax
from jax.experimental import pallas as pl
from jax.experimental.pallas import tpu as pltpu
from jax.experimental.pallas import tpu_sc as plsc

_N = 10000
_NPAD = 10240
_D = 128
_E = 320000
_NC = 2
_NS = 16
_NW = _NC * _NS
_CH = 128
_EW = 10240
_EPAD = _NW * _EW
_NCHUNK = _EW // _CH
_G = 8
_NG = _NCHUNK // _G
_RPT = _NPAD // _NS
_DEGW = 16
_DAUG = _D + _DEGW


def _zero_rows(rows, width):
  zv = jnp.zeros((16,), jnp.float32)
  def zrow(i, carry):
    for k in range(width // 16):
      rows[i, pl.ds(k * 16, 16)] = zv
    return carry
  lax.fori_loop(0, _CH, zrow, 0)


def _make_segsum(compute_deg):
  mesh = plsc.VectorSubcoreMesh(core_axis_name="c", subcore_axis_name="s")
  out_type = jax.ShapeDtypeStruct((_NC, _NPAD, _D), jnp.float32)
  scratch = (
      pltpu.VMEM_SHARED((_NPAD, _D), jnp.float32),
      pltpu.VMEM((_G, _CH), jnp.int32),
      pltpu.VMEM((_G, _CH), jnp.int32),
      pltpu.VMEM((_CH, _D), jnp.float32),
      pltpu.VMEM((_CH, _D), jnp.float32),
      pltpu.SemaphoreType.DMA,
      pltpu.SemaphoreType.DMA,
  )

  def body(feat, srcr, dstr, s_out, acc, srcv, dstv, rows, rows1, sem, sem1):
    bufs = (rows, rows1)
    sems = (sem, sem1)
    c = lax.axis_index("c")
    s = lax.axis_index("s")
    w = c * _NS + s
    r0 = s * _RPT
    nblk = _RPT // _CH

    _zero_rows(rows, _D)
    for t in range(nblk):
      pltpu.sync_copy(rows, acc.at[pl.ds(r0 + t * _CH, _CH)])
    if compute_deg:
      ov = jnp.ones((16,), jnp.float32)
      def onerow(i, carry):
        for k in range(_D // 16):
          rows[i, pl.ds(k * 16, 16)] = ov
        return carry
      lax.fori_loop(0, _CH, onerow, 0)
    plsc.subcore_barrier()

    def group(g, carry):
      if not compute_deg:
        pltpu.sync_copy(srcr.at[w, pl.ds(g * _G, _G)], srcv)
      pltpu.sync_copy(dstr.at[w, pl.ds(g * _G, _G)], dstv)
      if compute_deg:
        for j in range(_G):
          pltpu.sync_copy(rows, acc.at[dstv.at[j]], add=True)
        return carry
      desc = [pltpu.async_copy(feat.at[srcv.at[0]], bufs[0], sems[0]), None]
      for j in range(_G):
        if j + 1 < _G:
          desc[(j + 1) % 2] = pltpu.async_copy(
              feat.at[srcv.at[j + 1]], bufs[(j + 1) % 2], sems[(j + 1) % 2])
        desc[j % 2].wait()
        pltpu.sync_copy(bufs[j % 2], acc.at[dstv.at[j]], add=True)
      return carry

    lax.fori_loop(0, _NG, group, 0)
    plsc.subcore_barrier()
    for t in range(nblk):
      rb = r0 + t * _CH
      pltpu.sync_copy(acc.at[pl.ds(rb, _CH)], rows)
      pltpu.sync_copy(rows, s_out.at[c, pl.ds(rb, _CH)])

  return functools.partial(
      pl.kernel, mesh=mesh, out_type=out_type,
      scratch_types=scratch)(body)


_make_segsum = functools.lru_cache(maxsize=None)(_make_segsum)

_BN = 1024


def _dense_layer(S, deg, x, W_l, b_l, W_r, *, sigmoid):
  def body(s_ref, d_ref, x_ref, wl_ref, b_ref, wr_ref, o_ref):
    ssum = s_ref[0] + s_ref[1]
    agg = ssum / jnp.maximum(d_ref[0] + d_ref[1], 1.0)
    r = jnp.dot(agg, wl_ref[...], preferred_element_type=jnp.float32,
                precision=lax.Precision.HIGHEST)
    r += jnp.dot(x_ref[...], wr_ref[...], preferred_element_type=jnp.float32,
                 precision=lax.Precision.HIGHEST)
    r += b_ref[...]
    o_ref[...] = jax.nn.sigmoid(r) if sigmoid else jnp.maximum(r, 0.0)

  return pl.pallas_call(
      body,
      grid=(_NPAD // _BN,),
      in_specs=[
          pl.BlockSpec((2, _BN, _D), lambda i: (0, i, 0)),
          pl.BlockSpec((2, _BN, _D), lambda i: (0, i, 0)),
          pl.BlockSpec((_BN, _D), lambda i: (i, 0)),
          pl.BlockSpec((_D, _D), lambda i: (0, 0)),
          pl.BlockSpec((1, _D), lambda i: (0, 0)),
          pl.BlockSpec((_D, _D), lambda i: (0, 0)),
      ],
      out_specs=pl.BlockSpec((_BN, _D), lambda i: (i, 0)),
      out_shape=jax.ShapeDtypeStruct((_NPAD, _D), jnp.float32),
  )(S, deg, x, W_l, b_l.reshape(1, _D), W_r)


@jax.jit
def kernel(x, edge_index, W1_l, b1_l, W1_r, W2_l, b2_l, W2_r):
  src = edge_index[0]
  dst = edge_index[1]
  npad = _EPAD - _E
  pad_src = (jnp.arange(npad, dtype=jnp.int32) * 37) % _N
  pad_dst = _N + jnp.arange(npad, dtype=jnp.int32) % (_NPAD - _N)
  srcp = jnp.concatenate([src, pad_src]).reshape(_NW, _NCHUNK, _CH)
  dstp = jnp.concatenate([dst, pad_dst]).reshape(_NW, _NCHUNK, _CH)
  xpad = jnp.concatenate([x, jnp.zeros((_NPAD - _N, _D), x.dtype)])

  deg = _make_segsum(True)(xpad, srcp, dstp)
  S1 = _make_segsum(False)(xpad, srcp, dstp)
  h = _dense_layer(S1, deg, xpad, W1_l, b1_l, W1_r, sigmoid=False)
  S2 = _make_segsum(False)(h, srcp, dstp)
  out = _dense_layer(S2, deg, h, W2_l, b2_l, W2_r, sigmoid=True)
  return out[:_N]


# speedup vs baseline: 9.3977x; 1.0833x over previous
import functools

import jax
import jax.numpy as jnp
from jax import lax
from jax.experimental import pallas as pl
from jax.experimental.pallas import tpu as pltpu
from jax.experimental.pallas import tpu_sc as plsc

_N = 10000
_NPAD = 10240
_D = 128
_E = 320000
_NC = 2
_NS = 16
_NW = _NC * _NS
_CH = 128
_EW = 10240
_EPAD = _NW * _EW
_NCHUNK = _EW // _CH
_G = 8
_NG = _NCHUNK // _G
_RPT = _NPAD // _NS
_DEGW = 16
_DAUG = _D + _DEGW


def _zero_rows(rows, width):
  zv = jnp.zeros((16,), jnp.float32)
  def zrow(i, carry):
    for k in range(width // 16):
      rows[i, pl.ds(k * 16, 16)] = zv
    return carry
  lax.fori_loop(0, _CH, zrow, 0)


def _make_segsum(compute_deg):
  mesh = plsc.VectorSubcoreMesh(core_axis_name="c", subcore_axis_name="s")
  out_type = jax.ShapeDtypeStruct((_NC, _NPAD, _D), jnp.float32)
  scratch = (
      pltpu.VMEM_SHARED((_NPAD, _D), jnp.float32),
      pltpu.VMEM((2, _G, _CH), jnp.int32),
      pltpu.VMEM((2, _G, _CH), jnp.int32),
      pltpu.VMEM((_CH, _D), jnp.float32),
      pltpu.VMEM((_CH, _D), jnp.float32),
      pltpu.SemaphoreType.DMA,
      pltpu.SemaphoreType.DMA,
      pltpu.SemaphoreType.DMA,
  )

  def body(feat, srcr, dstr, s_out, acc, srcv, dstv, rows, rows1, sem, sem1,
           semidx):
    bufs = (rows, rows1)
    sems = (sem, sem1)
    c = lax.axis_index("c")
    s = lax.axis_index("s")
    w = c * _NS + s
    r0 = s * _RPT
    nblk = _RPT // _CH

    _zero_rows(rows, _D)
    for t in range(nblk):
      pltpu.sync_copy(rows, acc.at[pl.ds(r0 + t * _CH, _CH)])
    if compute_deg:
      ov = jnp.ones((16,), jnp.float32)
      def onerow(i, carry):
        for k in range(_D // 16):
          rows[i, pl.ds(k * 16, 16)] = ov
        return carry
      lax.fori_loop(0, _CH, onerow, 0)
    plsc.subcore_barrier()

    if not compute_deg:
      pltpu.sync_copy(srcr.at[w, pl.ds(0, _G)], srcv.at[0])
    pltpu.sync_copy(dstr.at[w, pl.ds(0, _G)], dstv.at[0])

    def group(g, carry):
      sl = lax.rem(g, 2)
      gn = jnp.minimum(g + 1, _NG - 1) * _G
      idx_descs = []
      if not compute_deg:
        idx_descs.append(pltpu.async_copy(
            srcr.at[w, pl.ds(gn, _G)], srcv.at[1 - sl], semidx))
      idx_descs.append(pltpu.async_copy(
          dstr.at[w, pl.ds(gn, _G)], dstv.at[1 - sl], semidx))
      if compute_deg:
        for j in range(_G):
          pltpu.sync_copy(rows, acc.at[dstv.at[sl, j]], add=True)
      else:
        desc = [pltpu.async_copy(feat.at[srcv.at[sl, 0]], bufs[0], sems[0]),
                None]
        for j in range(_G):
          if j + 1 < _G:
            desc[(j + 1) % 2] = pltpu.async_copy(
                feat.at[srcv.at[sl, j + 1]], bufs[(j + 1) % 2],
                sems[(j + 1) % 2])
          desc[j % 2].wait()
          pltpu.sync_copy(bufs[j % 2], acc.at[dstv.at[sl, j]], add=True)
      for d in idx_descs:
        d.wait()
      return carry

    lax.fori_loop(0, _NG, group, 0)
    plsc.subcore_barrier()
    for t in range(nblk):
      rb = r0 + t * _CH
      pltpu.sync_copy(acc.at[pl.ds(rb, _CH)], rows)
      pltpu.sync_copy(rows, s_out.at[c, pl.ds(rb, _CH)])

  return functools.partial(
      pl.kernel, mesh=mesh, out_type=out_type,
      scratch_types=scratch)(body)


_make_segsum = functools.lru_cache(maxsize=None)(_make_segsum)

_BN = 1024


def _linr(x, W, b):
  def body(x_ref, w_ref, b_ref, o_ref):
    o_ref[...] = jnp.dot(
        x_ref[...], w_ref[...], preferred_element_type=jnp.float32,
        precision=lax.Precision.HIGHEST) + b_ref[...]

  return pl.pallas_call(
      body,
      grid=(_NPAD // _BN,),
      in_specs=[
          pl.BlockSpec((_BN, _D), lambda i: (i, 0)),
          pl.BlockSpec((_D, _D), lambda i: (0, 0)),
          pl.BlockSpec((1, _D), lambda i: (0, 0)),
      ],
      out_specs=pl.BlockSpec((_BN, _D), lambda i: (i, 0)),
      out_shape=jax.ShapeDtypeStruct((_NPAD, _D), jnp.float32),
  )(x, W, b.reshape(1, _D))


def _dense_fin(S, dpart, xr, W_l, *, first):
  def body(s_ref, d_ref, xr_ref, wl_ref, *out_refs):
    ssum = s_ref[0] + s_ref[1]
    if first:
      o_ref, invd_ref = out_refs
      invd = 1.0 / jnp.maximum(d_ref[0] + d_ref[1], 1.0)
      invd_ref[...] = invd
    else:
      (o_ref,) = out_refs
      invd = d_ref[...]
    r = jnp.dot(ssum * invd, wl_ref[...], preferred_element_type=jnp.float32,
                precision=lax.Precision.HIGHEST) + xr_ref[...]
    o_ref[...] = jnp.maximum(r, 0.0) if first else jax.nn.sigmoid(r)

  d_spec = (pl.BlockSpec((2, _BN, _D), lambda i: (0, i, 0)) if first
            else pl.BlockSpec((_BN, _D), lambda i: (i, 0)))
  out_shape = [jax.ShapeDtypeStruct((_NPAD, _D), jnp.float32)]
  out_specs = [pl.BlockSpec((_BN, _D), lambda i: (i, 0))]
  if first:
    out_shape.append(jax.ShapeDtypeStruct((_NPAD, _D), jnp.float32))
    out_specs.append(pl.BlockSpec((_BN, _D), lambda i: (i, 0)))
  return pl.pallas_call(
      body,
      grid=(_NPAD // _BN,),
      in_specs=[
          pl.BlockSpec((2, _BN, _D), lambda i: (0, i, 0)),
          d_spec,
          pl.BlockSpec((_BN, _D), lambda i: (i, 0)),
          pl.BlockSpec((_D, _D), lambda i: (0, 0)),
      ],
      out_specs=out_specs,
      out_shape=out_shape,
  )(S, dpart, xr, W_l)


@jax.jit
def kernel(x, edge_index, W1_l, b1_l, W1_r, W2_l, b2_l, W2_r):
  src = edge_index[0]
  dst = edge_index[1]
  npad = _EPAD - _E
  pad_src = (jnp.arange(npad, dtype=jnp.int32) * 37) % _N
  pad_dst = _N + jnp.arange(npad, dtype=jnp.int32) % (_NPAD - _N)
  srcp = jnp.concatenate([src, pad_src]).reshape(_NW, _NCHUNK, _CH)
  dstp = jnp.concatenate([dst, pad_dst]).reshape(_NW, _NCHUNK, _CH)
  xpad = jnp.concatenate([x, jnp.zeros((_NPAD - _N, _D), x.dtype)])

  deg = _make_segsum(True)(xpad, srcp, dstp)
  S1 = _make_segsum(False)(xpad, srcp, dstp)
  xr1 = _linr(xpad, W1_r, b1_l)
  h, invd = _dense_fin(S1, deg, xr1, W1_l, first=True)
  S2 = _make_segsum(False)(h, srcp, dstp)
  xr2 = _linr(h, W2_r, b2_l)
  (out,) = _dense_fin(S2, invd, xr2, W2_l, first=False)
  return out[:_N]


# speedup vs baseline: 10.6458x; 1.1328x over previous
import functools

import jax
import jax.numpy as jnp
from jax import lax
from jax.experimental import pallas as pl
from jax.experimental.pallas import tpu as pltpu
from jax.experimental.pallas import tpu_sc as plsc

_N = 10000
_NPAD = 10240
_D = 128
_E = 320000
_NC = 2
_NS = 16
_NW = _NC * _NS
_CH = 128
_EW = 10240
_EPAD = _NW * _EW
_NCHUNK = _EW // _CH
_G = 8
_NG = _NCHUNK // _G
_RPT = _NPAD // _NS
_DEGW = 16
_DAUG = _D + _DEGW


def _zero_rows(rows, width):
  zv = jnp.zeros((16,), jnp.float32)
  def zrow(i, carry):
    for k in range(width // 16):
      rows[i, pl.ds(k * 16, 16)] = zv
    return carry
  lax.fori_loop(0, _CH, zrow, 0)


def _deg_hist():
  mesh = plsc.VectorSubcoreMesh(core_axis_name="c", subcore_axis_name="s")
  out_type = jax.ShapeDtypeStruct((_NC, _NPAD), jnp.float32)
  scratch = (
      pltpu.VMEM_SHARED((_NS, _NPAD), jnp.float32),
      pltpu.VMEM((2, _G, _CH), jnp.int32),
      pltpu.VMEM((_NPAD,), jnp.float32),
      pltpu.VMEM((_RPT,), jnp.float32),
      pltpu.VMEM((_RPT,), jnp.float32),
      pltpu.SemaphoreType.DMA,
  )

  def body(dstr, d_out, dall, dstv, hist, tmp, lacc, semidx):
    c = lax.axis_index("c")
    s = lax.axis_index("s")
    w = c * _NS + s
    zv = jnp.zeros((16,), jnp.float32)
    ones = jnp.ones((16,), jnp.float32)

    def zrow(i, carry):
      hist[pl.ds(i * 16, 16)] = zv
      return carry
    lax.fori_loop(0, _NPAD // 16, zrow, 0)

    pltpu.sync_copy(dstr.at[w, pl.ds(0, _G)], dstv.at[0])

    def group(g, carry):
      sl = lax.rem(g, 2)
      gn = jnp.minimum(g + 1, _NG - 1) * _G
      d = pltpu.async_copy(dstr.at[w, pl.ds(gn, _G)], dstv.at[1 - sl], semidx)
      for j in range(_G):
        for k in range(_CH // 16):
          iv = dstv[sl, j, pl.ds(k * 16, 16)]
          plsc.addupdate_scatter(hist, [iv], ones)
      d.wait()
      return carry
    lax.fori_loop(0, _NG, group, 0)

    pltpu.sync_copy(hist, dall.at[s])
    plsc.subcore_barrier()
    r0 = s * _RPT
    def zrow2(i, carry):
      lacc[pl.ds(i * 16, 16)] = zv
      return carry
    lax.fori_loop(0, _RPT // 16, zrow2, 0)
    def mrg(t, carry):
      pltpu.sync_copy(dall.at[t, pl.ds(r0, _RPT)], tmp)
      def addrow(i, carry2):
        lacc[pl.ds(i * 16, 16)] = lacc[pl.ds(i * 16, 16)] + tmp[pl.ds(i * 16, 16)]
        return carry2
      lax.fori_loop(0, _RPT // 16, addrow, 0)
      return carry
    lax.fori_loop(0, _NS, mrg, 0)
    pltpu.sync_copy(lacc, d_out.at[c, pl.ds(r0, _RPT)])

  return functools.partial(
      pl.kernel, mesh=mesh, out_type=out_type, scratch_types=scratch,
      compiler_params=pltpu.CompilerParams(needs_layout_passes=False))(body)


def _make_segsum(compute_deg):
  mesh = plsc.VectorSubcoreMesh(core_axis_name="c", subcore_axis_name="s")
  out_type = jax.ShapeDtypeStruct((_NC, _NPAD, _D), jnp.float32)
  scratch = (
      pltpu.VMEM_SHARED((_NPAD, _D), jnp.float32),
      pltpu.VMEM((2, _G, _CH), jnp.int32),
      pltpu.VMEM((2, _G, _CH), jnp.int32),
      pltpu.VMEM((_CH, _D), jnp.float32),
      pltpu.VMEM((_CH, _D), jnp.float32),
      pltpu.SemaphoreType.DMA,
      pltpu.SemaphoreType.DMA,
      pltpu.SemaphoreType.DMA,
  )

  def body(feat, srcr, dstr, s_out, acc, srcv, dstv, rows, rows1, sem, sem1,
           semidx):
    bufs = (rows, rows1)
    sems = (sem, sem1)
    c = lax.axis_index("c")
    s = lax.axis_index("s")
    w = c * _NS + s
    r0 = s * _RPT
    nblk = _RPT // _CH

    _zero_rows(rows, _D)
    for t in range(nblk):
      pltpu.sync_copy(rows, acc.at[pl.ds(r0 + t * _CH, _CH)])
    if compute_deg:
      ov = jnp.ones((16,), jnp.float32)
      def onerow(i, carry):
        for k in range(_D // 16):
          rows[i, pl.ds(k * 16, 16)] = ov
        return carry
      lax.fori_loop(0, _CH, onerow, 0)
    plsc.subcore_barrier()

    if not compute_deg:
      pltpu.sync_copy(srcr.at[w, pl.ds(0, _G)], srcv.at[0])
    pltpu.sync_copy(dstr.at[w, pl.ds(0, _G)], dstv.at[0])

    def group(g, carry):
      sl = lax.rem(g, 2)
      gn = jnp.minimum(g + 1, _NG - 1) * _G
      idx_descs = []
      if not compute_deg:
        idx_descs.append(pltpu.async_copy(
            srcr.at[w, pl.ds(gn, _G)], srcv.at[1 - sl], semidx))
      idx_descs.append(pltpu.async_copy(
          dstr.at[w, pl.ds(gn, _G)], dstv.at[1 - sl], semidx))
      if compute_deg:
        for j in range(_G):
          pltpu.sync_copy(rows, acc.at[dstv.at[sl, j]], add=True)
      else:
        desc = [pltpu.async_copy(feat.at[srcv.at[sl, 0]], bufs[0], sems[0]),
                None]
        for j in range(_G):
          if j + 1 < _G:
            desc[(j + 1) % 2] = pltpu.async_copy(
                feat.at[srcv.at[sl, j + 1]], bufs[(j + 1) % 2],
                sems[(j + 1) % 2])
          desc[j % 2].wait()
          pltpu.sync_copy(bufs[j % 2], acc.at[dstv.at[sl, j]], add=True)
      for d in idx_descs:
        d.wait()
      return carry

    lax.fori_loop(0, _NG, group, 0)
    plsc.subcore_barrier()
    for t in range(nblk):
      rb = r0 + t * _CH
      pltpu.sync_copy(acc.at[pl.ds(rb, _CH)], rows)
      pltpu.sync_copy(rows, s_out.at[c, pl.ds(rb, _CH)])

  return functools.partial(
      pl.kernel, mesh=mesh, out_type=out_type,
      scratch_types=scratch)(body)


_make_segsum = functools.lru_cache(maxsize=None)(_make_segsum)
_deg_hist = functools.lru_cache(maxsize=None)(_deg_hist)

_BN = 1024


def _linr(x, W, b):
  def body(x_ref, w_ref, b_ref, o_ref):
    o_ref[...] = jnp.dot(
        x_ref[...], w_ref[...], preferred_element_type=jnp.float32,
        precision=lax.Precision.HIGHEST) + b_ref[...]

  return pl.pallas_call(
      body,
      grid=(_NPAD // _BN,),
      in_specs=[
          pl.BlockSpec((_BN, _D), lambda i: (i, 0)),
          pl.BlockSpec((_D, _D), lambda i: (0, 0)),
          pl.BlockSpec((1, _D), lambda i: (0, 0)),
      ],
      out_specs=pl.BlockSpec((_BN, _D), lambda i: (i, 0)),
      out_shape=jax.ShapeDtypeStruct((_NPAD, _D), jnp.float32),
  )(x, W, b.reshape(1, _D))


def _dense_fin(S, dpart, xr, W_l, *, first):
  def body(s_ref, d_ref, xr_ref, wl_ref, *out_refs):
    ssum = s_ref[0] + s_ref[1]
    if first:
      o_ref, invd_ref = out_refs
      d = d_ref[0, :, 0] + d_ref[1, :, 0]
      invd = (1.0 / jnp.maximum(d, 1.0))[:, None]
      invd_ref[...] = jnp.broadcast_to(invd, o_ref.shape)
      invd = invd_ref[...]
    else:
      (o_ref,) = out_refs
      invd = d_ref[...]
    r = jnp.dot(ssum * invd, wl_ref[...], preferred_element_type=jnp.float32,
                precision=lax.Precision.HIGHEST) + xr_ref[...]
    o_ref[...] = jnp.maximum(r, 0.0) if first else jax.nn.sigmoid(r)

  d_spec = (pl.BlockSpec((2, _BN, _DEGW), lambda i: (0, i, 0)) if first
            else pl.BlockSpec((_BN, _D), lambda i: (i, 0)))
  out_shape = [jax.ShapeDtypeStruct((_NPAD, _D), jnp.float32)]
  out_specs = [pl.BlockSpec((_BN, _D), lambda i: (i, 0))]
  if first:
    out_shape.append(jax.ShapeDtypeStruct((_NPAD, _D), jnp.float32))
    out_specs.append(pl.BlockSpec((_BN, _D), lambda i: (i, 0)))
  return pl.pallas_call(
      body,
      grid=(_NPAD // _BN,),
      in_specs=[
          pl.BlockSpec((2, _BN, _D), lambda i: (0, i, 0)),
          d_spec,
          pl.BlockSpec((_BN, _D), lambda i: (i, 0)),
          pl.BlockSpec((_D, _D), lambda i: (0, 0)),
      ],
      out_specs=out_specs,
      out_shape=out_shape,
  )(S, dpart, xr, W_l)


@jax.jit
def kernel(x, edge_index, W1_l, b1_l, W1_r, W2_l, b2_l, W2_r):
  src = edge_index[0]
  dst = edge_index[1]
  npad = _EPAD - _E
  pad_src = (jnp.arange(npad, dtype=jnp.int32) * 37) % _N
  pad_dst = _N + jnp.arange(npad, dtype=jnp.int32) % (_NPAD - _N)
  srcp = jnp.concatenate([src, pad_src]).reshape(_NW, _NCHUNK, _CH)
  dstp = jnp.concatenate([dst, pad_dst]).reshape(_NW, _NCHUNK, _CH)
  xpad = jnp.concatenate([x, jnp.zeros((_NPAD - _N, _D), x.dtype)])

  deg2 = _deg_hist()(dstp)
  deg = jnp.broadcast_to(deg2[:, :, None], (_NC, _NPAD, _DEGW))
  S1 = _make_segsum(False)(xpad, srcp, dstp)
  xr1 = _linr(xpad, W1_r, b1_l)
  h, invd = _dense_fin(S1, deg, xr1, W1_l, first=True)
  S2 = _make_segsum(False)(h, srcp, dstp)
  xr2 = _linr(h, W2_r, b2_l)
  (out,) = _dense_fin(S2, invd, xr2, W2_l, first=False)
  return out[:_N]


# speedup vs baseline: 11.0692x; 1.0398x over previous
import functools

import jax
import jax.numpy as jnp
from jax import lax
from jax.experimental import pallas as pl
from jax.experimental.pallas import tpu as pltpu
from jax.experimental.pallas import tpu_sc as plsc

_N = 10000
_NPAD = 10240
_D = 128
_E = 320000
_NC = 2
_NS = 16
_NW = _NC * _NS
_CH = 128
_EW = 10240
_EPAD = _NW * _EW
_NCHUNK = _EW // _CH
_G = 16
_NG = _NCHUNK // _G
_RPT = _NPAD // _NS
_DEGW = 16
_DAUG = _D + _DEGW


def _zero_rows(rows, width):
  zv = jnp.zeros((16,), jnp.float32)
  def zrow(i, carry):
    for k in range(width // 16):
      rows[i, pl.ds(k * 16, 16)] = zv
    return carry
  lax.fori_loop(0, _CH, zrow, 0)


def _deg_hist():
  mesh = plsc.VectorSubcoreMesh(core_axis_name="c", subcore_axis_name="s")
  out_type = jax.ShapeDtypeStruct((_NC, _NPAD), jnp.float32)
  scratch = (
      pltpu.VMEM_SHARED((_NS, _NPAD), jnp.float32),
      pltpu.VMEM((2, _G, _CH), jnp.int32),
      pltpu.VMEM((_NPAD,), jnp.float32),
      pltpu.VMEM((_RPT,), jnp.float32),
      pltpu.VMEM((_RPT,), jnp.float32),
      pltpu.SemaphoreType.DMA,
  )

  def body(dstr, d_out, dall, dstv, hist, tmp, lacc, semidx):
    c = lax.axis_index("c")
    s = lax.axis_index("s")
    w = c * _NS + s
    zv = jnp.zeros((16,), jnp.float32)
    ones = jnp.ones((16,), jnp.float32)

    def zrow(i, carry):
      hist[pl.ds(i * 16, 16)] = zv
      return carry
    lax.fori_loop(0, _NPAD // 16, zrow, 0)

    pltpu.sync_copy(dstr.at[w, pl.ds(0, _G)], dstv.at[0])

    def group(g, carry):
      sl = lax.rem(g, 2)
      gn = jnp.minimum(g + 1, _NG - 1) * _G
      d = pltpu.async_copy(dstr.at[w, pl.ds(gn, _G)], dstv.at[1 - sl], semidx)
      for j in range(_G):
        for k in range(_CH // 16):
          iv = dstv[sl, j, pl.ds(k * 16, 16)]
          plsc.addupdate_scatter(hist, [iv], ones)
      d.wait()
      return carry
    lax.fori_loop(0, _NG, group, 0)

    pltpu.sync_copy(hist, dall.at[s])
    plsc.subcore_barrier()
    r0 = s * _RPT
    def zrow2(i, carry):
      lacc[pl.ds(i * 16, 16)] = zv
      return carry
    lax.fori_loop(0, _RPT // 16, zrow2, 0)
    def mrg(t, carry):
      pltpu.sync_copy(dall.at[t, pl.ds(r0, _RPT)], tmp)
      def addrow(i, carry2):
        lacc[pl.ds(i * 16, 16)] = lacc[pl.ds(i * 16, 16)] + tmp[pl.ds(i * 16, 16)]
        return carry2
      lax.fori_loop(0, _RPT // 16, addrow, 0)
      return carry
    lax.fori_loop(0, _NS, mrg, 0)
    pltpu.sync_copy(lacc, d_out.at[c, pl.ds(r0, _RPT)])

  return functools.partial(
      pl.kernel, mesh=mesh, out_type=out_type, scratch_types=scratch,
      compiler_params=pltpu.CompilerParams(needs_layout_passes=False))(body)


def _make_segsum(compute_deg):
  mesh = plsc.VectorSubcoreMesh(core_axis_name="c", subcore_axis_name="s")
  out_type = jax.ShapeDtypeStruct((_NC, _NPAD, _D), jnp.float32)
  scratch = (
      pltpu.VMEM_SHARED((_NPAD, _D), jnp.float32),
      pltpu.VMEM((2, _G, _CH), jnp.int32),
      pltpu.VMEM((2, _G, _CH), jnp.int32),
      pltpu.VMEM((_CH, _D), jnp.float32),
      pltpu.VMEM((_CH, _D), jnp.float32),
      pltpu.SemaphoreType.DMA,
      pltpu.SemaphoreType.DMA,
      pltpu.SemaphoreType.DMA,
      pltpu.SemaphoreType.DMA,
      pltpu.SemaphoreType.DMA,
  )

  def body(feat, srcr, dstr, s_out, acc, srcv, dstv, rows, rows1, sem, sem1,
           ssem, ssem1, semidx):
    bufs = (rows, rows1)
    sems = (sem, sem1)
    ssems = (ssem, ssem1)
    c = lax.axis_index("c")
    s = lax.axis_index("s")
    w = c * _NS + s
    r0 = s * _RPT
    nblk = _RPT // _CH

    _zero_rows(rows, _D)
    for t in range(nblk):
      pltpu.sync_copy(rows, acc.at[pl.ds(r0 + t * _CH, _CH)])
    if compute_deg:
      ov = jnp.ones((16,), jnp.float32)
      def onerow(i, carry):
        for k in range(_D // 16):
          rows[i, pl.ds(k * 16, 16)] = ov
        return carry
      lax.fori_loop(0, _CH, onerow, 0)
    plsc.subcore_barrier()

    if not compute_deg:
      pltpu.sync_copy(srcr.at[w, pl.ds(0, _G)], srcv.at[0])
    pltpu.sync_copy(dstr.at[w, pl.ds(0, _G)], dstv.at[0])

    def group(g, carry):
      sl = lax.rem(g, 2)
      gn = jnp.minimum(g + 1, _NG - 1) * _G
      idx_descs = []
      if not compute_deg:
        idx_descs.append(pltpu.async_copy(
            srcr.at[w, pl.ds(gn, _G)], srcv.at[1 - sl], semidx))
      idx_descs.append(pltpu.async_copy(
          dstr.at[w, pl.ds(gn, _G)], dstv.at[1 - sl], semidx))
      if compute_deg:
        for j in range(_G):
          pltpu.sync_copy(rows, acc.at[dstv.at[sl, j]], add=True)
      else:
        desc = [pltpu.async_copy(feat.at[srcv.at[sl, 0]], bufs[0], sems[0]),
                None]
        sdesc = [None, None]
        for j in range(_G):
          if j + 1 < _G:
            if sdesc[(j + 1) % 2] is not None:
              sdesc[(j + 1) % 2].wait()
              sdesc[(j + 1) % 2] = None
            desc[(j + 1) % 2] = pltpu.async_copy(
                feat.at[srcv.at[sl, j + 1]], bufs[(j + 1) % 2],
                sems[(j + 1) % 2])
          desc[j % 2].wait()
          sdesc[j % 2] = pltpu.async_copy(
              bufs[j % 2], acc.at[dstv.at[sl, j]], ssems[j % 2], add=True)
        for d in sdesc:
          if d is not None:
            d.wait()
      for d in idx_descs:
        d.wait()
      return carry

    lax.fori_loop(0, _NG, group, 0)
    plsc.subcore_barrier()
    for t in range(nblk):
      rb = r0 + t * _CH
      pltpu.sync_copy(acc.at[pl.ds(rb, _CH)], rows)
      pltpu.sync_copy(rows, s_out.at[c, pl.ds(rb, _CH)])

  return functools.partial(
      pl.kernel, mesh=mesh, out_type=out_type,
      scratch_types=scratch)(body)


_make_segsum = functools.lru_cache(maxsize=None)(_make_segsum)
_deg_hist = functools.lru_cache(maxsize=None)(_deg_hist)

_BN = 1024


def _linr(x, W, b):
  def body(x_ref, w_ref, b_ref, o_ref):
    o_ref[...] = jnp.dot(
        x_ref[...], w_ref[...], preferred_element_type=jnp.float32,
        precision=lax.Precision.HIGHEST) + b_ref[...]

  return pl.pallas_call(
      body,
      grid=(_NPAD // _BN,),
      in_specs=[
          pl.BlockSpec((_BN, _D), lambda i: (i, 0)),
          pl.BlockSpec((_D, _D), lambda i: (0, 0)),
          pl.BlockSpec((1, _D), lambda i: (0, 0)),
      ],
      out_specs=pl.BlockSpec((_BN, _D), lambda i: (i, 0)),
      out_shape=jax.ShapeDtypeStruct((_NPAD, _D), jnp.float32),
  )(x, W, b.reshape(1, _D))


def _dense_fin(S, dpart, xr, W_l, *, first):
  def body(s_ref, d_ref, xr_ref, wl_ref, *out_refs):
    ssum = s_ref[0] + s_ref[1]
    if first:
      o_ref, invd_ref = out_refs
      d = d_ref[0, :, 0] + d_ref[1, :, 0]
      invd = (1.0 / jnp.maximum(d, 1.0))[:, None]
      invd_ref[...] = jnp.broadcast_to(invd, invd_ref.shape)
    else:
      (o_ref,) = out_refs
      invd = d_ref[:, 0][:, None]
    r = jnp.dot(ssum * invd, wl_ref[...], preferred_element_type=jnp.float32,
                precision=lax.Precision.HIGHEST) + xr_ref[...]
    o_ref[...] = jnp.maximum(r, 0.0) if first else jax.nn.sigmoid(r)

  d_spec = (pl.BlockSpec((2, _BN, _DEGW), lambda i: (0, i, 0)) if first
            else pl.BlockSpec((_BN, _DEGW), lambda i: (i, 0)))
  out_shape = [jax.ShapeDtypeStruct((_NPAD, _D), jnp.float32)]
  out_specs = [pl.BlockSpec((_BN, _D), lambda i: (i, 0))]
  if first:
    out_shape.append(jax.ShapeDtypeStruct((_NPAD, _DEGW), jnp.float32))
    out_specs.append(pl.BlockSpec((_BN, _DEGW), lambda i: (i, 0)))
  return pl.pallas_call(
      body,
      grid=(_NPAD // _BN,),
      in_specs=[
          pl.BlockSpec((2, _BN, _D), lambda i: (0, i, 0)),
          d_spec,
          pl.BlockSpec((_BN, _D), lambda i: (i, 0)),
          pl.BlockSpec((_D, _D), lambda i: (0, 0)),
      ],
      out_specs=out_specs,
      out_shape=out_shape,
  )(S, dpart, xr, W_l)


@jax.jit
def kernel(x, edge_index, W1_l, b1_l, W1_r, W2_l, b2_l, W2_r):
  src = edge_index[0]
  dst = edge_index[1]
  npad = _EPAD - _E
  pad_src = (jnp.arange(npad, dtype=jnp.int32) * 37) % _N
  pad_dst = _N + jnp.arange(npad, dtype=jnp.int32) % (_NPAD - _N)
  srcp = jnp.concatenate([src, pad_src]).reshape(_NW, _NCHUNK, _CH)
  dstp = jnp.concatenate([dst, pad_dst]).reshape(_NW, _NCHUNK, _CH)
  xpad = jnp.concatenate([x, jnp.zeros((_NPAD - _N, _D), x.dtype)])

  deg2 = _deg_hist()(dstp)
  deg = jnp.broadcast_to(deg2[:, :, None], (_NC, _NPAD, _DEGW))
  S1 = _make_segsum(False)(xpad, srcp, dstp)
  xr1 = _linr(xpad, W1_r, b1_l)
  h, invd = _dense_fin(S1, deg, xr1, W1_l, first=True)
  S2 = _make_segsum(False)(h, srcp, dstp)
  xr2 = _linr(h, W2_r, b2_l)
  (out,) = _dense_fin(S2, invd, xr2, W2_l, first=False)
  return out[:_N]


# speedup vs baseline: 11.1959x; 1.0114x over previous
import functools

import jax
import jax.numpy as jnp
from jax import lax
from jax.experimental import pallas as pl
from jax.experimental.pallas import tpu as pltpu
from jax.experimental.pallas import tpu_sc as plsc

_N = 10000
_NPAD = 10240
_D = 128
_E = 320000
_NC = 2
_NS = 16
_NW = _NC * _NS
_CH = 128
_EW = 10240
_EPAD = _NW * _EW
_NCHUNK = _EW // _CH
_G = 16
_NG = _NCHUNK // _G
_RPT = _NPAD // _NS
_DEGW = 16
_DAUG = _D + _DEGW


def _zero_rows(rows, width):
  zv = jnp.zeros((16,), jnp.float32)
  def zrow(i, carry):
    for k in range(width // 16):
      rows[i, pl.ds(k * 16, 16)] = zv
    return carry
  lax.fori_loop(0, _CH, zrow, 0)


def _deg_hist():
  mesh = plsc.VectorSubcoreMesh(core_axis_name="c", subcore_axis_name="s")
  out_type = jax.ShapeDtypeStruct((_NC, _NPAD), jnp.float32)
  scratch = (
      pltpu.VMEM_SHARED((_NS, _NPAD), jnp.float32),
      pltpu.VMEM((2, _G, _CH), jnp.int32),
      pltpu.VMEM((_NPAD,), jnp.float32),
      pltpu.VMEM((_RPT,), jnp.float32),
      pltpu.VMEM((_RPT,), jnp.float32),
      pltpu.SemaphoreType.DMA,
  )

  def body(dstr, d_out, dall, dstv, hist, tmp, lacc, semidx):
    c = lax.axis_index("c")
    s = lax.axis_index("s")
    w = c * _NS + s
    zv = jnp.zeros((16,), jnp.float32)
    ones = jnp.ones((16,), jnp.float32)

    def zrow(i, carry):
      hist[pl.ds(i * 16, 16)] = zv
      return carry
    lax.fori_loop(0, _NPAD // 16, zrow, 0)

    pltpu.sync_copy(dstr.at[w, pl.ds(0, _G)], dstv.at[0])

    def group(g, carry):
      sl = lax.rem(g, 2)
      gn = jnp.minimum(g + 1, _NG - 1) * _G
      d = pltpu.async_copy(dstr.at[w, pl.ds(gn, _G)], dstv.at[1 - sl], semidx)
      for j in range(_G):
        for k in range(_CH // 16):
          iv = dstv[sl, j, pl.ds(k * 16, 16)]
          plsc.addupdate_scatter(hist, [iv], ones)
      d.wait()
      return carry
    lax.fori_loop(0, _NG, group, 0)

    pltpu.sync_copy(hist, dall.at[s])
    plsc.subcore_barrier()
    r0 = s * _RPT
    def zrow2(i, carry):
      lacc[pl.ds(i * 16, 16)] = zv
      return carry
    lax.fori_loop(0, _RPT // 16, zrow2, 0)
    def mrg(t, carry):
      pltpu.sync_copy(dall.at[t, pl.ds(r0, _RPT)], tmp)
      def addrow(i, carry2):
        lacc[pl.ds(i * 16, 16)] = lacc[pl.ds(i * 16, 16)] + tmp[pl.ds(i * 16, 16)]
        return carry2
      lax.fori_loop(0, _RPT // 16, addrow, 0)
      return carry
    lax.fori_loop(0, _NS, mrg, 0)
    pltpu.sync_copy(lacc, d_out.at[c, pl.ds(r0, _RPT)])

  return functools.partial(
      pl.kernel, mesh=mesh, out_type=out_type, scratch_types=scratch,
      compiler_params=pltpu.CompilerParams(needs_layout_passes=False))(body)


def _make_segsum(compute_deg):
  mesh = plsc.VectorSubcoreMesh(core_axis_name="c", subcore_axis_name="s")
  out_type = jax.ShapeDtypeStruct((_NC, _NPAD, _D), jnp.float32)
  scratch = (
      pltpu.VMEM_SHARED((_NPAD, _D), jnp.float32),
      pltpu.VMEM((2, _G, _CH), jnp.int32),
      pltpu.VMEM((2, _G, _CH), jnp.int32),
      pltpu.VMEM((_CH, _D), jnp.float32),
      pltpu.VMEM((_CH, _D), jnp.float32),
      pltpu.SemaphoreType.DMA,
      pltpu.SemaphoreType.DMA,
      pltpu.SemaphoreType.DMA,
      pltpu.SemaphoreType.DMA,
      pltpu.SemaphoreType.DMA,
  )

  def body(feat, srcr, dstr, s_out, acc, srcv, dstv, rows, rows1, sem, sem1,
           ssem, ssem1, semidx):
    bufs = (rows, rows1)
    sems = (sem, sem1)
    ssems = (ssem, ssem1)
    c = lax.axis_index("c")
    s = lax.axis_index("s")
    w = c * _NS + s
    r0 = s * _RPT
    nblk = _RPT // _CH

    _zero_rows(rows, _D)
    for t in range(nblk):
      pltpu.sync_copy(rows, acc.at[pl.ds(r0 + t * _CH, _CH)])
    if compute_deg:
      ov = jnp.ones((16,), jnp.float32)
      def onerow(i, carry):
        for k in range(_D // 16):
          rows[i, pl.ds(k * 16, 16)] = ov
        return carry
      lax.fori_loop(0, _CH, onerow, 0)
    plsc.subcore_barrier()

    if not compute_deg:
      pltpu.sync_copy(srcr.at[w, pl.ds(0, _G)], srcv.at[0])
    pltpu.sync_copy(dstr.at[w, pl.ds(0, _G)], dstv.at[0])

    def group(g, carry):
      sl = lax.rem(g, 2)
      gn = jnp.minimum(g + 1, _NG - 1) * _G
      idx_descs = []
      if not compute_deg:
        idx_descs.append(pltpu.async_copy(
            srcr.at[w, pl.ds(gn, _G)], srcv.at[1 - sl], semidx))
      idx_descs.append(pltpu.async_copy(
          dstr.at[w, pl.ds(gn, _G)], dstv.at[1 - sl], semidx))
      if compute_deg:
        for j in range(_G):
          pltpu.sync_copy(rows, acc.at[dstv.at[sl, j]], add=True)
      else:
        desc = [pltpu.async_copy(feat.at[srcv.at[sl, 0]], bufs[0], sems[0]),
                None]
        sdesc = [None, None]
        for j in range(_G):
          if j + 1 < _G:
            if sdesc[(j + 1) % 2] is not None:
              sdesc[(j + 1) % 2].wait()
              sdesc[(j + 1) % 2] = None
            desc[(j + 1) % 2] = pltpu.async_copy(
                feat.at[srcv.at[sl, j + 1]], bufs[(j + 1) % 2],
                sems[(j + 1) % 2])
          desc[j % 2].wait()
          sdesc[j % 2] = pltpu.async_copy(
              bufs[j % 2], acc.at[dstv.at[sl, j]], ssems[j % 2], add=True)
        for d in sdesc:
          if d is not None:
            d.wait()
      for d in idx_descs:
        d.wait()
      return carry

    lax.fori_loop(0, _NG, group, 0)
    plsc.subcore_barrier()
    for t in range(nblk):
      rb = r0 + t * _CH
      pltpu.sync_copy(acc.at[pl.ds(rb, _CH)], rows)
      pltpu.sync_copy(rows, s_out.at[c, pl.ds(rb, _CH)])

  return functools.partial(
      pl.kernel, mesh=mesh, out_type=out_type,
      scratch_types=scratch)(body)


_make_segsum = functools.lru_cache(maxsize=None)(_make_segsum)
_deg_hist = functools.lru_cache(maxsize=None)(_deg_hist)

_BN = 2048


def _linr(x, W, b):
  def body(x_ref, w_ref, b_ref, o_ref):
    o_ref[...] = jnp.dot(
        x_ref[...], w_ref[...], preferred_element_type=jnp.float32,
        precision=lax.Precision.HIGHEST) + b_ref[...]

  return pl.pallas_call(
      body,
      grid=(_NPAD // _BN,),
      in_specs=[
          pl.BlockSpec((_BN, _D), lambda i: (i, 0)),
          pl.BlockSpec((_D, _D), lambda i: (0, 0)),
          pl.BlockSpec((1, _D), lambda i: (0, 0)),
      ],
      out_specs=pl.BlockSpec((_BN, _D), lambda i: (i, 0)),
      out_shape=jax.ShapeDtypeStruct((_NPAD, _D), jnp.float32),
  )(x, W, b.reshape(1, _D))


def _dense_fin(S, dpart, xr, W_l, *, first):
  def body(s_ref, d_ref, xr_ref, wl_ref, *out_refs):
    ssum = s_ref[0] + s_ref[1]
    if first:
      o_ref, invd_ref = out_refs
      d = d_ref[0, :, 0] + d_ref[1, :, 0]
      invd = (1.0 / jnp.maximum(d, 1.0))[:, None]
      invd_ref[...] = jnp.broadcast_to(invd, invd_ref.shape)
    else:
      (o_ref,) = out_refs
      invd = d_ref[:, 0][:, None]
    r = jnp.dot(ssum * invd, wl_ref[...], preferred_element_type=jnp.float32,
                precision=lax.Precision.HIGHEST) + xr_ref[...]
    o_ref[...] = jnp.maximum(r, 0.0) if first else jax.nn.sigmoid(r)

  d_spec = (pl.BlockSpec((2, _BN, _DEGW), lambda i: (0, i, 0)) if first
            else pl.BlockSpec((_BN, _DEGW), lambda i: (i, 0)))
  out_shape = [jax.ShapeDtypeStruct((_NPAD, _D), jnp.float32)]
  out_specs = [pl.BlockSpec((_BN, _D), lambda i: (i, 0))]
  if first:
    out_shape.append(jax.ShapeDtypeStruct((_NPAD, _DEGW), jnp.float32))
    out_specs.append(pl.BlockSpec((_BN, _DEGW), lambda i: (i, 0)))
  return pl.pallas_call(
      body,
      grid=(_NPAD // _BN,),
      in_specs=[
          pl.BlockSpec((2, _BN, _D), lambda i: (0, i, 0)),
          d_spec,
          pl.BlockSpec((_BN, _D), lambda i: (i, 0)),
          pl.BlockSpec((_D, _D), lambda i: (0, 0)),
      ],
      out_specs=out_specs,
      out_shape=out_shape,
  )(S, dpart, xr, W_l)


@jax.jit
def kernel(x, edge_index, W1_l, b1_l, W1_r, W2_l, b2_l, W2_r):
  src = edge_index[0]
  dst = edge_index[1]
  npad = _EPAD - _E
  pad_src = (jnp.arange(npad, dtype=jnp.int32) * 37) % _N
  pad_dst = _N + jnp.arange(npad, dtype=jnp.int32) % (_NPAD - _N)
  srcp = jnp.concatenate([src, pad_src]).reshape(_NW, _NCHUNK, _CH)
  dstp = jnp.concatenate([dst, pad_dst]).reshape(_NW, _NCHUNK, _CH)
  xpad = jnp.concatenate([x, jnp.zeros((_NPAD - _N, _D), x.dtype)])

  xr1 = _linr(xpad, W1_r, b1_l)
  deg2 = _deg_hist()(dstp)
  deg = jnp.broadcast_to(deg2[:, :, None], (_NC, _NPAD, _DEGW))
  S1 = _make_segsum(False)(xpad, srcp, dstp)
  h, invd = _dense_fin(S1, deg, xr1, W1_l, first=True)
  xr2 = _linr(h, W2_r, b2_l)
  S2 = _make_segsum(False)(h, srcp, dstp)
  (out,) = _dense_fin(S2, invd, xr2, W2_l, first=False)
  return out[:_N]
